# baseline (speedup 1.0000x reference)
import functools

import jax
import jax.numpy as jnp
from jax import lax
from jax.experimental import pallas as pl
from jax.experimental.pallas import tpu as pltpu
from jax.experimental.pallas import tpu_sc as plsc

_B, _L, _D = 4096, 200, 64
_NC, _NS = 2, 16
_NW = _NC * _NS
_NBLK = _L * (_B // 128)
_PER_W = _NBLK // _NW
_BH = _B // 128

_mesh = plsc.VectorSubcoreMesh(core_axis_name="c", subcore_axis_name="s")


@functools.partial(
    pl.kernel,
    mesh=_mesh,
    out_type=jax.ShapeDtypeStruct((_L, 8, _BH, 8, 128), jnp.float32),
    scratch_types=[
        pltpu.VMEM((_PER_W, 128), jnp.int32),
        pltpu.VMEM((2, 128, _D), jnp.float32),
        pltpu.VMEM((2, 8, 8, 128), jnp.float32),
        pltpu.SemaphoreType.DMA,
        pltpu.SemaphoreType.DMA,
        pltpu.SemaphoreType.DMA,
    ],
    compiler_params=pltpu.CompilerParams(use_tc_tiling_on_sc=False,
                                         needs_layout_passes=False),
)
def _emb(idx_hbm, tab_hbm, out_hbm, idx_v, rows_v, trows_v, gsem, ssem0,
         ssem1):
    wid = lax.axis_index("s") * _NC + lax.axis_index("c")
    base = wid * _PER_W
    pltpu.sync_copy(idx_hbm.at[wid], idx_v)

    ssems = (ssem0, ssem1)
    lanes = lax.iota(jnp.int32, 16)
    ris = [b0 * 16 + lanes for b0 in range(8)]

    def fire_g(t, h):
        pltpu.async_copy(tab_hbm.at[idx_v.at[t]], rows_v.at[h], gsem)

    def drain_g(h):
        pltpu.make_async_copy(tab_hbm.at[idx_v.at[0]], rows_v.at[h],
                              gsem).wait()

    def transpose(h):
        @pl.loop(0, _D)
        def _(d):
            ci = jnp.zeros((16,), jnp.int32) + d
            dh = d // 8
            dl = d % 8
            for b0 in range(8):
                v = plsc.load_gather(rows_v.at[h], [ris[b0], ci])
                trows_v[h, dh, dl, pl.ds(b0 * 16, 16)] = v

    def fire_s(t, h):
        blk = base + t
        l = blk // _BH
        bh = blk % _BH
        pltpu.async_copy(trows_v.at[h], out_hbm.at[l, :, bh], ssems[h])

    def drain_s(h):
        pltpu.make_async_copy(trows_v.at[h], out_hbm.at[0, :, 0],
                              ssems[h]).wait()

    fire_g(0, 0)
    drain_g(0)
    fire_g(1, 1)
    transpose(0)
    fire_s(0, 0)
    drain_g(1)
    fire_g(2, 0)
    transpose(1)
    fire_s(1, 1)

    @pl.loop(2, _PER_W - 2, step=2)
    def _(t0):
        for p in range(2):
            t = t0 + p
            h = p
            drain_g(h)
            fire_g(t + 1, 1 - h)
            drain_s(h)
            transpose(h)
            fire_s(t, h)

    drain_g(0)
    fire_g(_PER_W - 1, 1)
    drain_s(0)
    transpose(0)
    fire_s(_PER_W - 2, 0)
    drain_g(1)
    drain_s(1)
    transpose(1)
    fire_s(_PER_W - 1, 1)
    drain_s(0)
    drain_s(1)


def kernel(word_indices, table):
    idx = word_indices.T.reshape(_NW, _PER_W, 128).astype(jnp.int32)
    out = _emb(idx, table)
    return out.transpose(2, 4, 0, 1, 3).reshape(_B, _L, _D)


# speedup vs baseline: 2.1586x; 2.1586x over previous
import functools

import jax
import jax.numpy as jnp
from jax import lax
from jax.experimental import pallas as pl
from jax.experimental.pallas import tpu as pltpu
from jax.experimental.pallas import tpu_sc as plsc

_B, _L, _D = 4096, 200, 64
_NC, _NS = 2, 16
_NW = _NC * _NS
_NBLK = _L * (_B // 128)
_PER_W = _NBLK // _NW
_BH = _B // 128

_mesh = plsc.VectorSubcoreMesh(core_axis_name="c", subcore_axis_name="s")


@functools.partial(
    pl.kernel,
    mesh=_mesh,
    out_type=jax.ShapeDtypeStruct((_L, 8, _BH, 8, 128), jnp.float32),
    scratch_types=[
        pltpu.VMEM((_PER_W, 128), jnp.int32),
        pltpu.VMEM((2, 128, _D), jnp.float32),
        pltpu.VMEM((2, 8, 8, 128), jnp.float32),
        pltpu.SemaphoreType.DMA,
        pltpu.SemaphoreType.DMA,
        pltpu.SemaphoreType.DMA,
    ],
    compiler_params=pltpu.CompilerParams(use_tc_tiling_on_sc=False,
                                         needs_layout_passes=False),
)
def _emb(idx_hbm, tab_hbm, out_hbm, idx_v, rows_v, trows_v, gsem, ssem0,
         ssem1):
    wid = lax.axis_index("s") * _NC + lax.axis_index("c")
    base = wid * _PER_W
    pltpu.sync_copy(idx_hbm.at[wid], idx_v)

    ssems = (ssem0, ssem1)
    lanes = lax.iota(jnp.int32, 16)
    ris = [b0 * 16 + lanes for b0 in range(8)]

    def fire_g(t, h):
        pltpu.async_copy(tab_hbm.at[idx_v.at[t]], rows_v.at[h], gsem)

    def drain_g(h):
        pltpu.make_async_copy(tab_hbm.at[idx_v.at[0]], rows_v.at[h],
                              gsem).wait()

    def transpose(h):
        @functools.partial(plsc.parallel_loop, 0, _D, unroll=8)
        def _(d):
            ci = jnp.zeros((16,), jnp.int32) + d
            dh = d // 8
            dl = d % 8
            for b0 in range(8):
                v = plsc.load_gather(rows_v.at[h], [ris[b0], ci])
                trows_v[h, dh, dl, pl.ds(b0 * 16, 16)] = v

    def fire_s(t, h):
        blk = base + t
        l = blk // _BH
        bh = blk % _BH
        pltpu.async_copy(trows_v.at[h], out_hbm.at[l, :, bh], ssems[h])

    def drain_s(h):
        pltpu.make_async_copy(trows_v.at[h], out_hbm.at[0, :, 0],
                              ssems[h]).wait()

    fire_g(0, 0)
    drain_g(0)
    fire_g(1, 1)
    transpose(0)
    fire_s(0, 0)
    drain_g(1)
    fire_g(2, 0)
    transpose(1)
    fire_s(1, 1)

    @pl.loop(2, _PER_W - 2, step=2)
    def _(t0):
        for p in range(2):
            t = t0 + p
            h = p
            drain_g(h)
            fire_g(t + 1, 1 - h)
            drain_s(h)
            transpose(h)
            fire_s(t, h)

    drain_g(0)
    fire_g(_PER_W - 1, 1)
    drain_s(0)
    transpose(0)
    fire_s(_PER_W - 2, 0)
    drain_g(1)
    drain_s(1)
    transpose(1)
    fire_s(_PER_W - 1, 1)
    drain_s(0)
    drain_s(1)


def kernel(word_indices, table):
    idx = word_indices.T.reshape(_NW, _PER_W, 128).astype(jnp.int32)
    out = _emb(idx, table)
    return out.transpose(2, 4, 0, 1, 3).reshape(_B, _L, _D)
